# Initial kernel scaffold; baseline (speedup 1.0000x reference)
import functools

import jax
import jax.numpy as jnp
from jax import lax
from jax.experimental import pallas as pl
from jax.experimental.pallas import tpu as pltpu
from jax.experimental.pallas import tpu_sc as plsc

N = 10000
E = 320000
D = 128
P = 65536

NC = 2
NS = 16
NW = NC * NS

EPW = E // NW
CH = 80
NCH = EPW // CH
RPS = N // NS
ZR = 125
DEGW = 16

PPW = P // NW
PCH = 128
NPCH = PPW // PCH

_mesh = plsc.VectorSubcoreMesh(core_axis_name="c", subcore_axis_name="s")


def _segsum_body(feat_hbm, src_hbm, dst_hbm, pout_hbm, dout_hbm,
                 acc_sh, deg_sh, src_v, dst_v, rows_v, ones_v, zbuf, degz, sem):
    cid = lax.axis_index("c")
    sid = lax.axis_index("s")
    wid = cid * NS + sid

    zv = jnp.zeros((16,), jnp.float32)
    ov = jnp.ones((16,), jnp.float32)

    @pl.loop(0, ZR)
    def _(i):
        for j in range(D // 16):
            zbuf[i, pl.ds(j * 16, 16)] = zv

    @pl.loop(0, RPS)
    def _(i):
        degz[i, :] = zv

    @pl.loop(0, CH)
    def _(i):
        ones_v[i, :] = ov

    for k in range(RPS // ZR):
        pltpu.sync_copy(zbuf, acc_sh.at[pl.ds(sid * RPS + k * ZR, ZR)])
    pltpu.sync_copy(degz, deg_sh.at[pl.ds(sid * RPS, RPS)])
    plsc.subcore_barrier()

    base0 = wid * EPW

    @pl.loop(0, NCH)
    def _(c):
        base = base0 + c * CH
        pltpu.sync_copy(src_hbm.at[pl.ds(base, CH)], src_v)
        pltpu.sync_copy(dst_hbm.at[pl.ds(base, CH)], dst_v)
        pltpu.async_copy(feat_hbm.at[src_v], rows_v, sem).wait()
        pltpu.sync_copy(rows_v, acc_sh.at[dst_v], add=True)
        pltpu.sync_copy(ones_v, deg_sh.at[dst_v], add=True)

    plsc.subcore_barrier()
    for k in range(RPS // ZR):
        sl = pl.ds(sid * RPS + k * ZR, ZR)
        pltpu.sync_copy(acc_sh.at[sl], pout_hbm.at[cid].at[sl])
    dsl = pl.ds(sid * RPS, RPS)
    pltpu.sync_copy(deg_sh.at[dsl], dout_hbm.at[cid].at[dsl])


def _sc_segsum(features, src, dst):
    f = pl.kernel(
        _segsum_body,
        out_type=(
            jax.ShapeDtypeStruct((NC, N, D), jnp.float32),
            jax.ShapeDtypeStruct((NC, N, DEGW), jnp.float32),
        ),
        mesh=_mesh,
        scratch_types=[
            pltpu.VMEM_SHARED((N, D), jnp.float32),
            pltpu.VMEM_SHARED((N, DEGW), jnp.float32),
            pltpu.VMEM((CH,), jnp.int32),
            pltpu.VMEM((CH,), jnp.int32),
            pltpu.VMEM((CH, D), jnp.float32),
            pltpu.VMEM((CH, DEGW), jnp.float32),
            pltpu.VMEM((ZR, D), jnp.float32),
            pltpu.VMEM((RPS, DEGW), jnp.float32),
            pltpu.SemaphoreType.DMA,
        ],
    )
    return f(features, src, dst)


def _pair_gather_body(en_hbm, p1_hbm, p2_hbm, o1_hbm, o2_hbm,
                      i1_v, i2_v, r1_v, r2_v, sem1, sem2):
    cid = lax.axis_index("c")
    sid = lax.axis_index("s")
    wid = cid * NS + sid
    base0 = wid * PPW

    @pl.loop(0, NPCH)
    def _(c):
        base = base0 + c * PCH
        sl = pl.ds(base, PCH)
        pltpu.sync_copy(p1_hbm.at[sl], i1_v)
        pltpu.sync_copy(p2_hbm.at[sl], i2_v)
        cp1 = pltpu.async_copy(en_hbm.at[i1_v], r1_v, sem1)
        cp2 = pltpu.async_copy(en_hbm.at[i2_v], r2_v, sem2)
        cp1.wait()
        pltpu.sync_copy(r1_v, o1_hbm.at[sl])
        cp2.wait()
        pltpu.sync_copy(r2_v, o2_hbm.at[sl])


def _sc_pair_gather(en, pair1, pair2):
    f = pl.kernel(
        _pair_gather_body,
        out_type=(
            jax.ShapeDtypeStruct((P, D), jnp.float32),
            jax.ShapeDtypeStruct((P, D), jnp.float32),
        ),
        mesh=_mesh,
        scratch_types=[
            pltpu.VMEM((PCH,), jnp.int32),
            pltpu.VMEM((PCH,), jnp.int32),
            pltpu.VMEM((PCH, D), jnp.float32),
            pltpu.VMEM((PCH, D), jnp.float32),
            pltpu.SemaphoreType.DMA,
            pltpu.SemaphoreType.DMA,
        ],
    )
    return f(en, pair1, pair2)


_RB = 2000


def _encode_body(f_ref, p_ref, d_ref, w_ref, o_ref):
    deg = jnp.sum(d_ref[0], axis=1) + jnp.sum(d_ref[1], axis=1)
    nsum = p_ref[0] + p_ref[1]
    mean = nsum / jnp.maximum(deg, 1.0)[:, None]
    w = w_ref[...]
    emb = jnp.dot(f_ref[...], w[:D], preferred_element_type=jnp.float32)
    emb = emb + jnp.dot(mean, w[D:], preferred_element_type=jnp.float32)
    emb = jnp.maximum(emb, 0.0)
    s = jnp.sum(emb * emb, axis=1, keepdims=True)
    o_ref[...] = emb * lax.rsqrt(jnp.maximum(s, 1e-16))


def _tc_encode(features, partials, degp, W):
    return pl.pallas_call(
        _encode_body,
        grid=(N // _RB,),
        in_specs=[
            pl.BlockSpec((_RB, D), lambda i: (i, 0)),
            pl.BlockSpec((NC, _RB, D), lambda i: (0, i, 0)),
            pl.BlockSpec((NC, _RB, DEGW), lambda i: (0, i, 0)),
            pl.BlockSpec((2 * D, D), lambda i: (0, 0)),
        ],
        out_specs=pl.BlockSpec((_RB, D), lambda i: (i, 0)),
        out_shape=jax.ShapeDtypeStruct((N, D), jnp.float32),
    )(features, partials, degp, W)


_SB = 8192


def _scores_body(g1_ref, g2_ref, o_ref):
    o_ref[...] = jnp.sum(g1_ref[...] * g2_ref[...], axis=1)


def _tc_scores(g1, g2):
    return pl.pallas_call(
        _scores_body,
        grid=(P // _SB,),
        in_specs=[
            pl.BlockSpec((_SB, D), lambda i: (i, 0)),
            pl.BlockSpec((_SB, D), lambda i: (i, 0)),
        ],
        out_specs=pl.BlockSpec((_SB,), lambda i: (i,)),
        out_shape=jax.ShapeDtypeStruct((P,), jnp.float32),
    )(g1, g2)


def kernel(features, adj_lists, pair1, pair2, W):
    src = adj_lists[0]
    dst = adj_lists[1]
    partials, degp = _sc_segsum(features, src, dst)
    en = _tc_encode(features, partials, degp, W)
    g1, g2 = _sc_pair_gather(en, pair1, pair2)
    return _tc_scores(g1, g2)


# speedup vs baseline: 3.7926x; 3.7926x over previous
import functools

import jax
import jax.numpy as jnp
from jax import lax
from jax.experimental import pallas as pl
from jax.experimental.pallas import tpu as pltpu
from jax.experimental.pallas import tpu_sc as plsc

N = 10000
E = 320000
D = 128
P = 65536

NC = 2
NS = 16
NW = NC * NS

ESS = E // NS
CH = 80
NCH = ESS // CH
NP = 10240
RPS = NP // NS
ZR = 128
DEGW = 16

PPW = P // NW
PCH = 128
NPCH = PPW // PCH

_mesh = plsc.VectorSubcoreMesh(core_axis_name="c", subcore_axis_name="s")


def _segsum_body(feat_hbm, src_hbm, dst_hbm, pout_hbm, dout_hbm,
                 acc_sh, deg_sh, src_v, dst_v, rows_v, ones_v, zbuf, zdeg, sem):
    cid = lax.axis_index("c")
    sid = lax.axis_index("s")
    base0 = sid * ESS

    zv = jnp.zeros((16,), jnp.float32)
    ov = jnp.ones((16,), jnp.float32)

    @pl.when(cid == 0)
    def _():
        @pl.loop(0, ZR)
        def _(i):
            for j in range(D // 16):
                zbuf[i, pl.ds(j * 16, 16)] = zv

        for k in range(RPS // ZR):
            pltpu.sync_copy(zbuf, acc_sh.at[pl.ds(sid * RPS + k * ZR, ZR)])
        plsc.subcore_barrier()

        @pl.loop(0, NCH)
        def _(c):
            base = base0 + c * CH
            pltpu.sync_copy(src_hbm.at[pl.ds(base, CH)], src_v)
            pltpu.async_copy(feat_hbm.at[src_v], rows_v, sem).wait()
            pltpu.sync_copy(dst_hbm.at[pl.ds(base, CH)], dst_v)
            pltpu.sync_copy(rows_v, acc_sh.at[dst_v], add=True)

        plsc.subcore_barrier()
        for k in range(RPS // ZR):
            sl = pl.ds(sid * RPS + k * ZR, ZR)
            pltpu.sync_copy(acc_sh.at[sl], pout_hbm.at[sl])

    @pl.when(cid == 1)
    def _():
        @pl.loop(0, CH // 16)
        def _(i):
            ones_v[pl.ds(i * 16, 16)] = ov

        @pl.loop(0, RPS // 16)
        def _(i):
            zdeg[pl.ds(i * 16, 16)] = zv

        pltpu.sync_copy(zdeg, deg_sh.at[pl.ds(sid * RPS, RPS)])
        plsc.subcore_barrier()

        @pl.loop(0, NCH)
        def _(c):
            base = base0 + c * CH
            pltpu.sync_copy(dst_hbm.at[pl.ds(base, CH)], dst_v)
            pltpu.sync_copy(ones_v, deg_sh.at[dst_v], add=True)

        plsc.subcore_barrier()
        dsl = pl.ds(sid * RPS, RPS)
        pltpu.sync_copy(deg_sh.at[dsl], dout_hbm.at[dsl])


def _sc_segsum(features, src, dst):
    f = pl.kernel(
        _segsum_body,
        out_type=(
            jax.ShapeDtypeStruct((NP, D), jnp.float32),
            jax.ShapeDtypeStruct((NP,), jnp.float32),
        ),
        mesh=_mesh,
        scratch_types=[
            pltpu.VMEM_SHARED((NP, D), jnp.float32),
            pltpu.VMEM_SHARED((NP,), jnp.float32),
            pltpu.VMEM((CH,), jnp.int32),
            pltpu.VMEM((CH,), jnp.int32),
            pltpu.VMEM((CH, D), jnp.float32),
            pltpu.VMEM((CH,), jnp.float32),
            pltpu.VMEM((ZR, D), jnp.float32),
            pltpu.VMEM((RPS,), jnp.float32),
            pltpu.SemaphoreType.DMA,
        ],
    )
    return f(features, src, dst)


def _pair_gather_body(en_hbm, p1_hbm, p2_hbm, o1_hbm, o2_hbm,
                      i1_v, i2_v, r1_v, r2_v, sem1, sem2):
    cid = lax.axis_index("c")
    sid = lax.axis_index("s")
    wid = cid * NS + sid
    base0 = wid * PPW

    @pl.loop(0, NPCH)
    def _(c):
        base = base0 + c * PCH
        sl = pl.ds(base, PCH)
        pltpu.sync_copy(p1_hbm.at[sl], i1_v)
        pltpu.sync_copy(p2_hbm.at[sl], i2_v)
        cp1 = pltpu.async_copy(en_hbm.at[i1_v], r1_v, sem1)
        cp2 = pltpu.async_copy(en_hbm.at[i2_v], r2_v, sem2)
        cp1.wait()
        pltpu.sync_copy(r1_v, o1_hbm.at[sl])
        cp2.wait()
        pltpu.sync_copy(r2_v, o2_hbm.at[sl])


def _sc_pair_gather(en, pair1, pair2):
    f = pl.kernel(
        _pair_gather_body,
        out_type=(
            jax.ShapeDtypeStruct((P, D), jnp.float32),
            jax.ShapeDtypeStruct((P, D), jnp.float32),
        ),
        mesh=_mesh,
        scratch_types=[
            pltpu.VMEM((PCH,), jnp.int32),
            pltpu.VMEM((PCH,), jnp.int32),
            pltpu.VMEM((PCH, D), jnp.float32),
            pltpu.VMEM((PCH, D), jnp.float32),
            pltpu.SemaphoreType.DMA,
            pltpu.SemaphoreType.DMA,
        ],
    )
    return f(en, pair1, pair2)


_RB = 2048


def _encode_body(f_ref, p_ref, d_ref, w_ref, o_ref):
    mean = p_ref[...] / jnp.maximum(d_ref[...], 1.0)[:, None]
    w = w_ref[...]
    emb = jnp.dot(f_ref[...], w[:D], preferred_element_type=jnp.float32)
    emb = emb + jnp.dot(mean, w[D:], preferred_element_type=jnp.float32)
    emb = jnp.maximum(emb, 0.0)
    s = jnp.sum(emb * emb, axis=1, keepdims=True)
    o_ref[...] = emb * lax.rsqrt(jnp.maximum(s, 1e-16))


def _tc_encode(features, partials, degp, W):
    return pl.pallas_call(
        _encode_body,
        grid=(NP // _RB,),
        in_specs=[
            pl.BlockSpec((_RB, D), lambda i: (i, 0)),
            pl.BlockSpec((_RB, D), lambda i: (i, 0)),
            pl.BlockSpec((_RB,), lambda i: (i,)),
            pl.BlockSpec((2 * D, D), lambda i: (0, 0)),
        ],
        out_specs=pl.BlockSpec((_RB, D), lambda i: (i, 0)),
        out_shape=jax.ShapeDtypeStruct((NP, D), jnp.float32),
    )(features, partials, degp, W)


_SB = 8192


def _scores_body(g1_ref, g2_ref, o_ref):
    o_ref[...] = jnp.sum(g1_ref[...] * g2_ref[...], axis=1)


def _tc_scores(g1, g2):
    return pl.pallas_call(
        _scores_body,
        grid=(P // _SB,),
        in_specs=[
            pl.BlockSpec((_SB, D), lambda i: (i, 0)),
            pl.BlockSpec((_SB, D), lambda i: (i, 0)),
        ],
        out_specs=pl.BlockSpec((_SB,), lambda i: (i,)),
        out_shape=jax.ShapeDtypeStruct((P,), jnp.float32),
    )(g1, g2)


def kernel(features, adj_lists, pair1, pair2, W):
    src = adj_lists[0]
    dst = adj_lists[1]
    partials, degp = _sc_segsum(features, src, dst)
    features_p = jnp.pad(features, ((0, NP - N), (0, 0)))
    en = _tc_encode(features_p, partials, degp, W)
    g1, g2 = _sc_pair_gather(en, pair1, pair2)
    return _tc_scores(g1, g2)


# speedup vs baseline: 7.9428x; 2.0943x over previous
import functools

import jax
import jax.numpy as jnp
from jax import lax
from jax.experimental import pallas as pl
from jax.experimental.pallas import tpu as pltpu
from jax.experimental.pallas import tpu_sc as plsc

N = 10000
E = 320000
D = 128
P = 65536

NC = 2
NS = 16
NW = NC * NS

ESS = E // NS
CH = 80
IB = 4000
NIB = ESS // IB
IBCH = IB // CH
NP = 10240
RPS = NP // NS
ZR = 40
DEGW = 16

PPW = P // NW
PCH = 128
NPCH = PPW // PCH

_mesh = plsc.VectorSubcoreMesh(core_axis_name="c", subcore_axis_name="s")


def _segsum_body(feat_hbm, src_hbm, dst_hbm, pout_hbm, dout_hbm,
                 acc_sh, src_blk, dst_blk, dv0, dv1, sv0, sv1, r0, r1,
                 zbuf, deg_l, sem0, sem1):
    cid = lax.axis_index("c")
    sid = lax.axis_index("s")
    ebase = sid * ESS

    zv = jnp.zeros((16,), jnp.float32)

    rows = (r0, r1)
    dvs = (dv0, dv1)
    svs = (sv0, sv1)
    sems = (sem0, sem1)

    def stage_idx(dv, blk, c, n):
        for j in range(n // 16):
            dv[pl.ds(j * 16, 16)] = blk[pl.ds(c * n + j * 16, 16)]

    @pl.when(cid == 0)
    def _():
        @pl.loop(0, ZR)
        def _(i):
            for j in range(D // 16):
                zbuf[i, pl.ds(j * 16, 16)] = zv

        for k in range(RPS // ZR):
            pltpu.sync_copy(zbuf, acc_sh.at[pl.ds(sid * RPS + k * ZR, ZR)])
        plsc.subcore_barrier()

        @pl.loop(0, NIB)
        def _(ib):
            eb = ebase + ib * IB
            pltpu.sync_copy(src_hbm.at[pl.ds(eb, IB)], src_blk)
            pltpu.sync_copy(dst_hbm.at[pl.ds(eb, IB)], dst_blk)

            for b in range(2):
                stage_idx(dvs[b], dst_blk, b, CH)
                stage_idx(svs[b], src_blk, b, CH)
                pltpu.async_copy(feat_hbm.at[svs[b]], rows[b], sems[b])

            @pl.loop(0, (IBCH - 2) // 2)
            def _(i):
                for b in range(2):
                    c = 2 * i + b
                    pltpu.make_async_copy(
                        feat_hbm.at[pl.ds(0, CH)], rows[b], sems[b]).wait()
                    pltpu.sync_copy(rows[b], acc_sh.at[dvs[b]], add=True)
                    stage_idx(dvs[b], dst_blk, c + 2, CH)
                    stage_idx(svs[b], src_blk, c + 2, CH)
                    pltpu.async_copy(feat_hbm.at[svs[b]], rows[b], sems[b])

            for b in range(2):
                pltpu.make_async_copy(
                    feat_hbm.at[pl.ds(0, CH)], rows[b], sems[b]).wait()
                pltpu.sync_copy(rows[b], acc_sh.at[dvs[b]], add=True)

        plsc.subcore_barrier()
        for k in range(RPS // ZR):
            sl = pl.ds(sid * RPS + k * ZR, ZR)
            pltpu.sync_copy(acc_sh.at[sl], pout_hbm.at[sl])

    @pl.when(cid == 1)
    def _():
        @pl.loop(0, NP // 16)
        def _(i):
            deg_l[pl.ds(i * 16, 16)] = zv

        @pl.loop(0, NIB)
        def _(ib):
            eb = ebase + ib * IB
            pltpu.sync_copy(dst_hbm.at[pl.ds(eb, IB)], dst_blk)

            @pl.loop(0, IB // 16)
            def _(g):
                d16 = dst_blk[pl.ds(g * 16, 16)]
                cnt_f, _ = plsc.scan_count(d16)
                cnt_r, _ = plsc.scan_count(lax.rev(d16, (0,)))
                last = lax.rev(cnt_r, (0,)) == 1
                plsc.addupdate_scatter(deg_l, [d16],
                                       cnt_f.astype(jnp.float32),
                                       mask=last)

        pltpu.sync_copy(deg_l, dout_hbm.at[sid])


def _sc_segsum(features, src, dst):
    f = pl.kernel(
        _segsum_body,
        out_type=(
            jax.ShapeDtypeStruct((NP, D), jnp.float32),
            jax.ShapeDtypeStruct((NS, NP), jnp.float32),
        ),
        mesh=_mesh,
        scratch_types=[
            pltpu.VMEM_SHARED((NP, D), jnp.float32),
            pltpu.VMEM((IB,), jnp.int32),
            pltpu.VMEM((IB,), jnp.int32),
            pltpu.VMEM((CH,), jnp.int32),
            pltpu.VMEM((CH,), jnp.int32),
            pltpu.VMEM((CH,), jnp.int32),
            pltpu.VMEM((CH,), jnp.int32),
            pltpu.VMEM((CH, D), jnp.float32),
            pltpu.VMEM((CH, D), jnp.float32),
            pltpu.VMEM((ZR, D), jnp.float32),
            pltpu.VMEM((NP,), jnp.float32),
            pltpu.SemaphoreType.DMA,
            pltpu.SemaphoreType.DMA,
        ],
        compiler_params=pltpu.CompilerParams(needs_layout_passes=False),
    )
    return f(features, src, dst)


def _pair_gather_body(en_hbm, p1_hbm, p2_hbm, o1_hbm, o2_hbm,
                      i1_all, i2_all, j1a, j1b, j2a, j2b,
                      r1a, r1b, r2a, r2b, s1a, s1b, s2a, s2b):
    cid = lax.axis_index("c")
    sid = lax.axis_index("s")
    wid = cid * NS + sid
    base0 = wid * PPW

    j1 = (j1a, j1b)
    j2 = (j2a, j2b)
    r1 = (r1a, r1b)
    r2 = (r2a, r2b)
    s1 = (s1a, s1b)
    s2 = (s2a, s2b)

    pltpu.sync_copy(p1_hbm.at[pl.ds(base0, PPW)], i1_all)
    pltpu.sync_copy(p2_hbm.at[pl.ds(base0, PPW)], i2_all)

    def stage(jv, blk, c):
        for j in range(PCH // 16):
            jv[pl.ds(j * 16, 16)] = blk[pl.ds(c * PCH + j * 16, 16)]

    cp1 = [None] * NPCH
    cp2 = [None] * NPCH
    for c in range(2):
        b = c % 2
        stage(j1[b], i1_all, c)
        cp1[c] = pltpu.async_copy(en_hbm.at[j1[b]], r1[b], s1[b])
        stage(j2[b], i2_all, c)
        cp2[c] = pltpu.async_copy(en_hbm.at[j2[b]], r2[b], s2[b])
    for c in range(NPCH):
        b = c % 2
        sl = pl.ds(base0 + c * PCH, PCH)
        cp1[c].wait()
        pltpu.sync_copy(r1[b], o1_hbm.at[sl])
        if c + 2 < NPCH:
            stage(j1[b], i1_all, c + 2)
            cp1[c + 2] = pltpu.async_copy(en_hbm.at[j1[b]], r1[b], s1[b])
        cp2[c].wait()
        pltpu.sync_copy(r2[b], o2_hbm.at[sl])
        if c + 2 < NPCH:
            stage(j2[b], i2_all, c + 2)
            cp2[c + 2] = pltpu.async_copy(en_hbm.at[j2[b]], r2[b], s2[b])


def _sc_pair_gather(en, pair1, pair2):
    f = pl.kernel(
        _pair_gather_body,
        out_type=(
            jax.ShapeDtypeStruct((P, D), jnp.float32),
            jax.ShapeDtypeStruct((P, D), jnp.float32),
        ),
        mesh=_mesh,
        scratch_types=[
            pltpu.VMEM((PPW,), jnp.int32),
            pltpu.VMEM((PPW,), jnp.int32),
            pltpu.VMEM((PCH,), jnp.int32),
            pltpu.VMEM((PCH,), jnp.int32),
            pltpu.VMEM((PCH,), jnp.int32),
            pltpu.VMEM((PCH,), jnp.int32),
            pltpu.VMEM((PCH, D), jnp.float32),
            pltpu.VMEM((PCH, D), jnp.float32),
            pltpu.VMEM((PCH, D), jnp.float32),
            pltpu.VMEM((PCH, D), jnp.float32),
            pltpu.SemaphoreType.DMA,
            pltpu.SemaphoreType.DMA,
            pltpu.SemaphoreType.DMA,
            pltpu.SemaphoreType.DMA,
        ],
    )
    return f(en, pair1, pair2)


_RB = 2048


def _encode_body(f_ref, p_ref, d_ref, w_ref, o_ref):
    deg = jnp.sum(d_ref[...], axis=0)
    mean = p_ref[...] / jnp.maximum(deg, 1.0)[:, None]
    w = w_ref[...]
    emb = jnp.dot(f_ref[...], w[:D], preferred_element_type=jnp.float32)
    emb = emb + jnp.dot(mean, w[D:], preferred_element_type=jnp.float32)
    emb = jnp.maximum(emb, 0.0)
    s = jnp.sum(emb * emb, axis=1, keepdims=True)
    o_ref[...] = emb * lax.rsqrt(jnp.maximum(s, 1e-16))


def _tc_encode(features, partials, degp, W):
    return pl.pallas_call(
        _encode_body,
        grid=(NP // _RB,),
        in_specs=[
            pl.BlockSpec((_RB, D), lambda i: (i, 0)),
            pl.BlockSpec((_RB, D), lambda i: (i, 0)),
            pl.BlockSpec((NS, _RB), lambda i: (0, i)),
            pl.BlockSpec((2 * D, D), lambda i: (0, 0)),
        ],
        out_specs=pl.BlockSpec((_RB, D), lambda i: (i, 0)),
        out_shape=jax.ShapeDtypeStruct((NP, D), jnp.float32),
    )(features, partials, degp, W)


_SB = 8192


def _scores_body(g1_ref, g2_ref, o_ref):
    o_ref[...] = jnp.sum(g1_ref[...] * g2_ref[...], axis=1)


def _tc_scores(g1, g2):
    return pl.pallas_call(
        _scores_body,
        grid=(P // _SB,),
        in_specs=[
            pl.BlockSpec((_SB, D), lambda i: (i, 0)),
            pl.BlockSpec((_SB, D), lambda i: (i, 0)),
        ],
        out_specs=pl.BlockSpec((_SB,), lambda i: (i,)),
        out_shape=jax.ShapeDtypeStruct((P,), jnp.float32),
    )(g1, g2)


def kernel(features, adj_lists, pair1, pair2, W):
    src = adj_lists[0]
    dst = adj_lists[1]
    partials, degp = _sc_segsum(features, src, dst)
    features_p = jnp.pad(features, ((0, NP - N), (0, 0)))
    en = _tc_encode(features_p, partials, degp, W)
    g1, g2 = _sc_pair_gather(en, pair1, pair2)
    return _tc_scores(g1, g2)


# speedup vs baseline: 9.8476x; 1.2398x over previous
import functools

import jax
import jax.numpy as jnp
from jax import lax
from jax.experimental import pallas as pl
from jax.experimental.pallas import tpu as pltpu
from jax.experimental.pallas import tpu_sc as plsc

N = 10000
E = 320000
D = 128
P = 65536

NC = 2
NS = 16
NW = NC * NS

ESS = E // NS
CH = 80
IB = 4000
NIB = ESS // IB
NP = 10240
HN = NP // NC
TR = 256
HNP = HN + TR
RPSH = HNP // NS
OPS = HN // NS
ZR = 48
DEGW = 16

PPW = P // NW
PCH = 128
NPCH = PPW // PCH

_mesh = plsc.VectorSubcoreMesh(core_axis_name="c", subcore_axis_name="s")


def _segsum_body(feat_hbm, src_hbm, dst_hbm, pout_hbm, dout_hbm,
                 acc_sh, src_blk, dst_blk, stg_s, stg_d, dv0, dv1, sv0, sv1,
                 r0, r1, zbuf, deg_l, zdeg, sem0, sem1):
    cid = lax.axis_index("c")
    sid = lax.axis_index("s")
    ebase = sid * ESS
    lo = cid * HN

    zv = jnp.zeros((16,), jnp.float32)
    lane = lax.iota(jnp.int32, 16)

    rows = (r0, r1)
    dvs = (dv0, dv1)
    svs = (sv0, sv1)
    sems = (sem0, sem1)

    def stage_idx(dv, blk, c):
        for j in range(CH // 16):
            dv[pl.ds(j * 16, 16)] = blk[pl.ds(c * CH + j * 16, 16)]

    @pl.loop(0, ZR)
    def _(i):
        for j in range(D // 16):
            zbuf[i, pl.ds(j * 16, 16)] = zv

    for k in range(RPSH // ZR):
        pltpu.sync_copy(zbuf, acc_sh.at[pl.ds(sid * RPSH + k * ZR, ZR)])

    @pl.loop(0, HNP // 16)
    def _(i):
        deg_l[pl.ds(i * 16, 16)] = zv

    @pl.loop(0, HN // 16)
    def _(i):
        zdeg[pl.ds(i * 16, 16)] = zv

    plsc.subcore_barrier()

    @pl.loop(0, NIB)
    def _(ib):
        eb = ebase + ib * IB
        pltpu.sync_copy(src_hbm.at[pl.ds(eb, IB)], src_blk)
        pltpu.sync_copy(dst_hbm.at[pl.ds(eb, IB)], dst_blk)

        def grp(g, n):
            s16 = src_blk[pl.ds(g * 16, 16)]
            d16 = dst_blk[pl.ds(g * 16, 16)]
            l16 = d16 - lo
            m = (l16 >= 0) & (l16 < HN)
            lsafe = jnp.where(m, l16, HN + (lane & (TR - 1)))
            plsc.store_compressed(stg_d.at[pl.ds(n, 16)], l16, mask=m)
            plsc.store_compressed(stg_s.at[pl.ds(n, 16)], s16, mask=m)
            cf, _ = plsc.scan_count(lsafe)
            cr, _ = plsc.scan_count(lax.rev(lsafe, (0,)))
            last = lax.rev(cr, (0,)) == 1
            plsc.addupdate_scatter(deg_l, [lsafe], cf.astype(jnp.float32),
                                   mask=last)
            return n + jnp.sum(jnp.where(m, 1, 0))

        n = lax.fori_loop(0, IB // 16, grp, jnp.int32(0))

        nch = (n + CH - 1) // CH
        npair = (nch + 1) // 2
        ntot = npair * (2 * CH)
        g0 = n // 16

        def padg(k, _):
            off = (g0 + k) * 16
            al = lane + off
            keep = al < n
            kd = stg_d[pl.ds(off, 16)]
            ks = stg_s[pl.ds(off, 16)]
            stg_d[pl.ds(off, 16)] = jnp.where(keep, kd, HN + (al & (TR - 1)))
            stg_s[pl.ds(off, 16)] = jnp.where(keep, ks, al & 1023)
            return k

        lax.fori_loop(0, (ntot - g0 * 16) // 16, padg, jnp.int32(0))

        @pl.when(npair >= 1)
        def _():
            for b in range(2):
                stage_idx(dvs[b], stg_d, b)
                stage_idx(svs[b], stg_s, b)
                pltpu.async_copy(feat_hbm.at[svs[b]], rows[b], sems[b])

        def pair_body(i, _):
            for b in range(2):
                c = 2 * i + b
                pltpu.make_async_copy(
                    feat_hbm.at[pl.ds(0, CH)], rows[b], sems[b]).wait()
                pltpu.sync_copy(rows[b], acc_sh.at[dvs[b]], add=True)
                stage_idx(dvs[b], stg_d, c + 2)
                stage_idx(svs[b], stg_s, c + 2)
                pltpu.async_copy(feat_hbm.at[svs[b]], rows[b], sems[b])
            return i

        lax.fori_loop(0, jnp.maximum(npair - 1, 0), pair_body, jnp.int32(0))

        @pl.when(npair >= 1)
        def _():
            for b in range(2):
                pltpu.make_async_copy(
                    feat_hbm.at[pl.ds(0, CH)], rows[b], sems[b]).wait()
                pltpu.sync_copy(rows[b], acc_sh.at[dvs[b]], add=True)

    plsc.subcore_barrier()
    pltpu.sync_copy(acc_sh.at[pl.ds(sid * OPS, OPS)],
                    pout_hbm.at[pl.ds(lo + sid * OPS, OPS)])
    wrow = cid * NS + sid
    pltpu.sync_copy(deg_l.at[pl.ds(0, HN)],
                    dout_hbm.at[wrow].at[pl.ds(lo, HN)])
    pltpu.sync_copy(zdeg, dout_hbm.at[wrow].at[pl.ds(HN - lo, HN)])


def _sc_segsum(features, src, dst):
    f = pl.kernel(
        _segsum_body,
        out_type=(
            jax.ShapeDtypeStruct((NP, D), jnp.float32),
            jax.ShapeDtypeStruct((NW, NP), jnp.float32),
        ),
        mesh=_mesh,
        scratch_types=[
            pltpu.VMEM_SHARED((HNP, D), jnp.float32),
            pltpu.VMEM((IB,), jnp.int32),
            pltpu.VMEM((IB,), jnp.int32),
            pltpu.VMEM((IB + 16,), jnp.int32),
            pltpu.VMEM((IB + 16,), jnp.int32),
            pltpu.VMEM((CH,), jnp.int32),
            pltpu.VMEM((CH,), jnp.int32),
            pltpu.VMEM((CH,), jnp.int32),
            pltpu.VMEM((CH,), jnp.int32),
            pltpu.VMEM((CH, D), jnp.float32),
            pltpu.VMEM((CH, D), jnp.float32),
            pltpu.VMEM((ZR, D), jnp.float32),
            pltpu.VMEM((HNP,), jnp.float32),
            pltpu.VMEM((HN,), jnp.float32),
            pltpu.SemaphoreType.DMA,
            pltpu.SemaphoreType.DMA,
        ],
        compiler_params=pltpu.CompilerParams(needs_layout_passes=False),
    )
    return f(features, src, dst)


def _pair_gather_body(en_hbm, p1_hbm, p2_hbm, o1_hbm, o2_hbm,
                      i1_all, i2_all, j1a, j1b, j2a, j2b,
                      r1a, r1b, r2a, r2b, s1a, s1b, s2a, s2b):
    cid = lax.axis_index("c")
    sid = lax.axis_index("s")
    wid = cid * NS + sid
    base0 = wid * PPW

    j1 = (j1a, j1b)
    j2 = (j2a, j2b)
    r1 = (r1a, r1b)
    r2 = (r2a, r2b)
    s1 = (s1a, s1b)
    s2 = (s2a, s2b)

    pltpu.sync_copy(p1_hbm.at[pl.ds(base0, PPW)], i1_all)
    pltpu.sync_copy(p2_hbm.at[pl.ds(base0, PPW)], i2_all)

    def stage(jv, blk, c):
        for j in range(PCH // 16):
            jv[pl.ds(j * 16, 16)] = blk[pl.ds(c * PCH + j * 16, 16)]

    cp1 = [None] * NPCH
    cp2 = [None] * NPCH
    for c in range(2):
        b = c % 2
        stage(j1[b], i1_all, c)
        cp1[c] = pltpu.async_copy(en_hbm.at[j1[b]], r1[b], s1[b])
        stage(j2[b], i2_all, c)
        cp2[c] = pltpu.async_copy(en_hbm.at[j2[b]], r2[b], s2[b])
    for c in range(NPCH):
        b = c % 2
        sl = pl.ds(base0 + c * PCH, PCH)
        cp1[c].wait()
        pltpu.sync_copy(r1[b], o1_hbm.at[sl])
        if c + 2 < NPCH:
            stage(j1[b], i1_all, c + 2)
            cp1[c + 2] = pltpu.async_copy(en_hbm.at[j1[b]], r1[b], s1[b])
        cp2[c].wait()
        pltpu.sync_copy(r2[b], o2_hbm.at[sl])
        if c + 2 < NPCH:
            stage(j2[b], i2_all, c + 2)
            cp2[c + 2] = pltpu.async_copy(en_hbm.at[j2[b]], r2[b], s2[b])


def _sc_pair_gather(en, pair1, pair2):
    f = pl.kernel(
        _pair_gather_body,
        out_type=(
            jax.ShapeDtypeStruct((P, D), jnp.float32),
            jax.ShapeDtypeStruct((P, D), jnp.float32),
        ),
        mesh=_mesh,
        scratch_types=[
            pltpu.VMEM((PPW,), jnp.int32),
            pltpu.VMEM((PPW,), jnp.int32),
            pltpu.VMEM((PCH,), jnp.int32),
            pltpu.VMEM((PCH,), jnp.int32),
            pltpu.VMEM((PCH,), jnp.int32),
            pltpu.VMEM((PCH,), jnp.int32),
            pltpu.VMEM((PCH, D), jnp.float32),
            pltpu.VMEM((PCH, D), jnp.float32),
            pltpu.VMEM((PCH, D), jnp.float32),
            pltpu.VMEM((PCH, D), jnp.float32),
            pltpu.SemaphoreType.DMA,
            pltpu.SemaphoreType.DMA,
            pltpu.SemaphoreType.DMA,
            pltpu.SemaphoreType.DMA,
        ],
    )
    return f(en, pair1, pair2)


_RB = 2048


def _encode_body(f_ref, p_ref, d_ref, w_ref, o_ref):
    deg = jnp.sum(d_ref[...], axis=0)
    mean = p_ref[...] / jnp.maximum(deg, 1.0)[:, None]
    w = w_ref[...]
    emb = jnp.dot(f_ref[...], w[:D], preferred_element_type=jnp.float32)
    emb = emb + jnp.dot(mean, w[D:], preferred_element_type=jnp.float32)
    emb = jnp.maximum(emb, 0.0)
    s = jnp.sum(emb * emb, axis=1, keepdims=True)
    o_ref[...] = emb * lax.rsqrt(jnp.maximum(s, 1e-16))


def _tc_encode(features, partials, degp, W):
    return pl.pallas_call(
        _encode_body,
        grid=(NP // _RB,),
        in_specs=[
            pl.BlockSpec((_RB, D), lambda i: (i, 0)),
            pl.BlockSpec((_RB, D), lambda i: (i, 0)),
            pl.BlockSpec((NW, _RB), lambda i: (0, i)),
            pl.BlockSpec((2 * D, D), lambda i: (0, 0)),
        ],
        out_specs=pl.BlockSpec((_RB, D), lambda i: (i, 0)),
        out_shape=jax.ShapeDtypeStruct((NP, D), jnp.float32),
    )(features, partials, degp, W)


_SB = 8192


def _scores_body(g1_ref, g2_ref, o_ref):
    o_ref[...] = jnp.sum(g1_ref[...] * g2_ref[...], axis=1)


def _tc_scores(g1, g2):
    return pl.pallas_call(
        _scores_body,
        grid=(P // _SB,),
        in_specs=[
            pl.BlockSpec((_SB, D), lambda i: (i, 0)),
            pl.BlockSpec((_SB, D), lambda i: (i, 0)),
        ],
        out_specs=pl.BlockSpec((_SB,), lambda i: (i,)),
        out_shape=jax.ShapeDtypeStruct((P,), jnp.float32),
    )(g1, g2)


def kernel(features, adj_lists, pair1, pair2, W):
    src = adj_lists[0]
    dst = adj_lists[1]
    partials, degp = _sc_segsum(features, src, dst)
    features_p = jnp.pad(features, ((0, NP - N), (0, 0)))
    en = _tc_encode(features_p, partials, degp, W)
    g1, g2 = _sc_pair_gather(en, pair1, pair2)
    return _tc_scores(g1, g2)


# speedup vs baseline: 10.0212x; 1.0176x over previous
import functools

import jax
import jax.numpy as jnp
from jax import lax
from jax.experimental import pallas as pl
from jax.experimental.pallas import tpu as pltpu
from jax.experimental.pallas import tpu_sc as plsc

N = 10000
E = 320000
D = 128
P = 65536

NC = 2
NS = 16
NW = NC * NS

ESS = E // NS
CH = 80
IB = 4000
NIB = ESS // IB
NP = 10240
HN = NP // NC
TR = 256
HNP = HN + TR
RPSH = HNP // NS
OPS = HN // NS
ZR = 48
DEGW = 16

PPW = P // NW
PCH = 128
NPCH = PPW // PCH

_mesh = plsc.VectorSubcoreMesh(core_axis_name="c", subcore_axis_name="s")


def _segsum_body(feat_hbm, src_hbm, dst_hbm, pout_hbm, dout_hbm,
                 acc_sh, src_blk, dst_blk, src_blk2, dst_blk2, stg_s, stg_d,
                 dv0, dv1, sv0, sv1,
                 r0, r1, zbuf, deg_l, zdeg, sem0, sem1, bsem0, bsem1):
    cid = lax.axis_index("c")
    sid = lax.axis_index("s")
    ebase = sid * ESS
    lo = cid * HN

    zv = jnp.zeros((16,), jnp.float32)
    lane = lax.iota(jnp.int32, 16)

    rows = (r0, r1)
    dvs = (dv0, dv1)
    svs = (sv0, sv1)
    sems = (sem0, sem1)

    def stage_idx(dv, blk, c):
        for j in range(CH // 16):
            dv[pl.ds(j * 16, 16)] = blk[pl.ds(c * CH + j * 16, 16)]

    @pl.loop(0, ZR)
    def _(i):
        for j in range(D // 16):
            zbuf[i, pl.ds(j * 16, 16)] = zv

    for k in range(RPSH // ZR):
        pltpu.sync_copy(zbuf, acc_sh.at[pl.ds(sid * RPSH + k * ZR, ZR)])

    @pl.loop(0, HNP // 16)
    def _(i):
        deg_l[pl.ds(i * 16, 16)] = zv

    @pl.loop(0, HN // 16)
    def _(i):
        zdeg[pl.ds(i * 16, 16)] = zv

    plsc.subcore_barrier()

    sblks = (src_blk, src_blk2)
    dblks = (dst_blk, dst_blk2)
    bsems = (bsem0, bsem1)

    pltpu.async_copy(src_hbm.at[pl.ds(ebase, IB)], src_blk, bsem0)
    pltpu.async_copy(dst_hbm.at[pl.ds(ebase, IB)], dst_blk, bsem0)

    for ib in range(NIB):
        p = ib % 2
        q = (ib + 1) % 2
        sb, db = sblks[p], dblks[p]
        pltpu.make_async_copy(src_hbm.at[pl.ds(0, IB)], sb, bsems[p]).wait()
        pltpu.make_async_copy(dst_hbm.at[pl.ds(0, IB)], db, bsems[p]).wait()
        if ib + 1 < NIB:
            eb2 = ebase + (ib + 1) * IB
            pltpu.async_copy(src_hbm.at[pl.ds(eb2, IB)], sblks[q], bsems[q])
            pltpu.async_copy(dst_hbm.at[pl.ds(eb2, IB)], dblks[q], bsems[q])

        def grp(g, n):
            s16 = sb[pl.ds(g * 16, 16)]
            d16 = db[pl.ds(g * 16, 16)]
            l16 = d16 - lo
            m = (l16 >= 0) & (l16 < HN)
            lsafe = jnp.where(m, l16, HN + (lane & (TR - 1)))
            plsc.store_compressed(stg_d.at[pl.ds(n, 16)], l16, mask=m)
            plsc.store_compressed(stg_s.at[pl.ds(n, 16)], s16, mask=m)
            cf, _ = plsc.scan_count(lsafe)
            cr, _ = plsc.scan_count(lax.rev(lsafe, (0,)))
            last = lax.rev(cr, (0,)) == 1
            plsc.addupdate_scatter(deg_l, [lsafe], cf.astype(jnp.float32),
                                   mask=last)
            return n + jnp.sum(jnp.where(m, 1, 0))

        n = lax.fori_loop(0, IB // 16, grp, jnp.int32(0))

        nch = (n + CH - 1) // CH
        npair = (nch + 1) // 2
        ntot = npair * (2 * CH)
        g0 = n // 16

        def padg(k, _):
            off = (g0 + k) * 16
            al = lane + off
            keep = al < n
            kd = stg_d[pl.ds(off, 16)]
            ks = stg_s[pl.ds(off, 16)]
            stg_d[pl.ds(off, 16)] = jnp.where(keep, kd, HN + (al & (TR - 1)))
            stg_s[pl.ds(off, 16)] = jnp.where(keep, ks, al & 1023)
            return k

        lax.fori_loop(0, (ntot - g0 * 16) // 16, padg, jnp.int32(0))

        @pl.when(npair >= 1)
        def _():
            for b in range(2):
                stage_idx(dvs[b], stg_d, b)
                stage_idx(svs[b], stg_s, b)
                pltpu.async_copy(feat_hbm.at[svs[b]], rows[b], sems[b])

        def pair_body(i, _):
            for b in range(2):
                c = 2 * i + b
                pltpu.make_async_copy(
                    feat_hbm.at[pl.ds(0, CH)], rows[b], sems[b]).wait()
                pltpu.sync_copy(rows[b], acc_sh.at[dvs[b]], add=True)
                stage_idx(dvs[b], stg_d, c + 2)
                stage_idx(svs[b], stg_s, c + 2)
                pltpu.async_copy(feat_hbm.at[svs[b]], rows[b], sems[b])
            return i

        lax.fori_loop(0, jnp.maximum(npair - 1, 0), pair_body, jnp.int32(0))

        @pl.when(npair >= 1)
        def _():
            for b in range(2):
                pltpu.make_async_copy(
                    feat_hbm.at[pl.ds(0, CH)], rows[b], sems[b]).wait()
                pltpu.sync_copy(rows[b], acc_sh.at[dvs[b]], add=True)

    plsc.subcore_barrier()
    pltpu.sync_copy(acc_sh.at[pl.ds(sid * OPS, OPS)],
                    pout_hbm.at[pl.ds(lo + sid * OPS, OPS)])
    wrow = cid * NS + sid
    pltpu.sync_copy(deg_l.at[pl.ds(0, HN)],
                    dout_hbm.at[wrow].at[pl.ds(lo, HN)])
    pltpu.sync_copy(zdeg, dout_hbm.at[wrow].at[pl.ds(HN - lo, HN)])


def _sc_segsum(features, src, dst):
    f = pl.kernel(
        _segsum_body,
        out_type=(
            jax.ShapeDtypeStruct((NP, D), jnp.float32),
            jax.ShapeDtypeStruct((NW, NP), jnp.float32),
        ),
        mesh=_mesh,
        scratch_types=[
            pltpu.VMEM_SHARED((HNP, D), jnp.float32),
            pltpu.VMEM((IB,), jnp.int32),
            pltpu.VMEM((IB,), jnp.int32),
            pltpu.VMEM((IB,), jnp.int32),
            pltpu.VMEM((IB,), jnp.int32),
            pltpu.VMEM((IB + 16,), jnp.int32),
            pltpu.VMEM((IB + 16,), jnp.int32),
            pltpu.VMEM((CH,), jnp.int32),
            pltpu.VMEM((CH,), jnp.int32),
            pltpu.VMEM((CH,), jnp.int32),
            pltpu.VMEM((CH,), jnp.int32),
            pltpu.VMEM((CH, D), jnp.float32),
            pltpu.VMEM((CH, D), jnp.float32),
            pltpu.VMEM((ZR, D), jnp.float32),
            pltpu.VMEM((HNP,), jnp.float32),
            pltpu.VMEM((HN,), jnp.float32),
            pltpu.SemaphoreType.DMA,
            pltpu.SemaphoreType.DMA,
            pltpu.SemaphoreType.DMA,
            pltpu.SemaphoreType.DMA,
        ],
        compiler_params=pltpu.CompilerParams(needs_layout_passes=False),
    )
    return f(features, src, dst)


def _pair_gather_body(en_hbm, p1_hbm, p2_hbm, o1_hbm, o2_hbm,
                      i1_all, i2_all, j1a, j1b, j2a, j2b,
                      r1a, r1b, r2a, r2b, s1a, s1b, s2a, s2b):
    cid = lax.axis_index("c")
    sid = lax.axis_index("s")
    wid = cid * NS + sid
    base0 = wid * PPW

    j1 = (j1a, j1b)
    j2 = (j2a, j2b)
    r1 = (r1a, r1b)
    r2 = (r2a, r2b)
    s1 = (s1a, s1b)
    s2 = (s2a, s2b)

    pltpu.sync_copy(p1_hbm.at[pl.ds(base0, PPW)], i1_all)
    pltpu.sync_copy(p2_hbm.at[pl.ds(base0, PPW)], i2_all)

    def stage(jv, blk, c):
        for j in range(PCH // 16):
            jv[pl.ds(j * 16, 16)] = blk[pl.ds(c * PCH + j * 16, 16)]

    cp1 = [None] * NPCH
    cp2 = [None] * NPCH
    for c in range(2):
        b = c % 2
        stage(j1[b], i1_all, c)
        cp1[c] = pltpu.async_copy(en_hbm.at[j1[b]], r1[b], s1[b])
        stage(j2[b], i2_all, c)
        cp2[c] = pltpu.async_copy(en_hbm.at[j2[b]], r2[b], s2[b])
    for c in range(NPCH):
        b = c % 2
        sl = pl.ds(base0 + c * PCH, PCH)
        cp1[c].wait()
        pltpu.sync_copy(r1[b], o1_hbm.at[sl])
        if c + 2 < NPCH:
            stage(j1[b], i1_all, c + 2)
            cp1[c + 2] = pltpu.async_copy(en_hbm.at[j1[b]], r1[b], s1[b])
        cp2[c].wait()
        pltpu.sync_copy(r2[b], o2_hbm.at[sl])
        if c + 2 < NPCH:
            stage(j2[b], i2_all, c + 2)
            cp2[c + 2] = pltpu.async_copy(en_hbm.at[j2[b]], r2[b], s2[b])


def _sc_pair_gather(en, pair1, pair2):
    f = pl.kernel(
        _pair_gather_body,
        out_type=(
            jax.ShapeDtypeStruct((P, D), jnp.float32),
            jax.ShapeDtypeStruct((P, D), jnp.float32),
        ),
        mesh=_mesh,
        scratch_types=[
            pltpu.VMEM((PPW,), jnp.int32),
            pltpu.VMEM((PPW,), jnp.int32),
            pltpu.VMEM((PCH,), jnp.int32),
            pltpu.VMEM((PCH,), jnp.int32),
            pltpu.VMEM((PCH,), jnp.int32),
            pltpu.VMEM((PCH,), jnp.int32),
            pltpu.VMEM((PCH, D), jnp.float32),
            pltpu.VMEM((PCH, D), jnp.float32),
            pltpu.VMEM((PCH, D), jnp.float32),
            pltpu.VMEM((PCH, D), jnp.float32),
            pltpu.SemaphoreType.DMA,
            pltpu.SemaphoreType.DMA,
            pltpu.SemaphoreType.DMA,
            pltpu.SemaphoreType.DMA,
        ],
    )
    return f(en, pair1, pair2)


_RB = 2048


def _encode_body(f_ref, p_ref, d_ref, w_ref, o_ref):
    deg = jnp.sum(d_ref[...], axis=0)
    mean = p_ref[...] / jnp.maximum(deg, 1.0)[:, None]
    w = w_ref[...]
    emb = jnp.dot(f_ref[...], w[:D], preferred_element_type=jnp.float32)
    emb = emb + jnp.dot(mean, w[D:], preferred_element_type=jnp.float32)
    emb = jnp.maximum(emb, 0.0)
    s = jnp.sum(emb * emb, axis=1, keepdims=True)
    o_ref[...] = emb * lax.rsqrt(jnp.maximum(s, 1e-16))


def _tc_encode(features, partials, degp, W):
    return pl.pallas_call(
        _encode_body,
        grid=(NP // _RB,),
        in_specs=[
            pl.BlockSpec((_RB, D), lambda i: (i, 0)),
            pl.BlockSpec((_RB, D), lambda i: (i, 0)),
            pl.BlockSpec((NW, _RB), lambda i: (0, i)),
            pl.BlockSpec((2 * D, D), lambda i: (0, 0)),
        ],
        out_specs=pl.BlockSpec((_RB, D), lambda i: (i, 0)),
        out_shape=jax.ShapeDtypeStruct((NP, D), jnp.float32),
    )(features, partials, degp, W)


_SB = 8192


def _scores_body(g1_ref, g2_ref, o_ref):
    o_ref[...] = jnp.sum(g1_ref[...] * g2_ref[...], axis=1)


def _tc_scores(g1, g2):
    return pl.pallas_call(
        _scores_body,
        grid=(P // _SB,),
        in_specs=[
            pl.BlockSpec((_SB, D), lambda i: (i, 0)),
            pl.BlockSpec((_SB, D), lambda i: (i, 0)),
        ],
        out_specs=pl.BlockSpec((_SB,), lambda i: (i,)),
        out_shape=jax.ShapeDtypeStruct((P,), jnp.float32),
    )(g1, g2)


def kernel(features, adj_lists, pair1, pair2, W):
    src = adj_lists[0]
    dst = adj_lists[1]
    partials, degp = _sc_segsum(features, src, dst)
    features_p = jnp.pad(features, ((0, NP - N), (0, 0)))
    en = _tc_encode(features_p, partials, degp, W)
    g1, g2 = _sc_pair_gather(en, pair1, pair2)
    return _tc_scores(g1, g2)
